# baseline (speedup 1.0000x reference)
import jax
import jax.numpy as jnp
from jax import lax
from jax.experimental import pallas as pl
from jax.experimental.pallas import tpu as pltpu
from jax.experimental.pallas import tpu_sc as plsc

_OUT_SHAPE = (8, 224, 224, 96)
_OUT_SIZE = 8 * 224 * 224 * 96
_N = 8 * 112 * 112 * 96

_NC = 2
_NS = 16
_NW = _NC * _NS
_R = _OUT_SIZE // _NW
_PLANE = 224 * 96
_NPLANES = _R // _PLANE
_T = _PLANE
_W = 16384
_NVREG = _N // 16
_I32_MIN = -(2**31)


def _load_window(sp, sx, spbuf, sxbuf, wb):
    wba = pl.multiple_of(wb, 16)
    last = wb >= _N - _W

    @pl.when(jnp.logical_not(last))
    def _full():
        pltpu.sync_copy(sp.at[pl.ds(wba, _W + 16)], spbuf)

    @pl.when(last)
    def _tail():
        pltpu.sync_copy(sp.at[pl.ds(wba, _W)], spbuf.at[pl.ds(0, _W)])
        spbuf[pl.ds(_W, 16)] = jnp.full((16,), -1, jnp.int32)

    pltpu.sync_copy(sx.at[pl.ds(wba, _W)], sxbuf)


def _scatter_body(sp, sx, out, bsbuf, spbuf, sxbuf, tile0, tile1, sem0, sem1):
    c = lax.axis_index("c")
    s = lax.axis_index("s")
    wid = s * _NC + c
    out_base = wid * _R

    iota16 = lax.iota(jnp.int32, 16)

    def bs_body(_, lohi):
        lo, hi = lohi
        mid = (lo + hi) // 2
        mid8 = jnp.minimum(mid & jnp.int32(~7), _N - 16)
        pltpu.sync_copy(sp.at[pl.ds(pl.multiple_of(mid8, 8), 16)], bsbuf)
        v16 = bsbuf[pl.ds(0, 16)]
        v = jnp.max(jnp.where(iota16 == mid - mid8, v16, _I32_MIN))
        go_right = v < out_base
        return (jnp.where(go_right, mid + 1, lo), jnp.where(go_right, hi, mid))

    a_lo, _ = lax.fori_loop(0, 24, bs_body, (jnp.int32(0), jnp.int32(_N)))

    wb0 = a_lo & jnp.int32(~(_W - 1))
    _load_window(sp, sx, spbuf, sxbuf, wb0)

    zeros16 = jnp.zeros((16,), jnp.float32)
    tiles = (tile0, tile1)
    sems = (sem0, sem1)

    def merge_tile(tile, tile_base, g, wb):
        tile_end = tile_base + _T

        def zbody(j, _):
            for u in range(6):
                tile[j, pl.ds(u * 16, 16)] = zeros16
            return 0

        lax.fori_loop(0, 224, zbody, 0)

        def wcond(cry):
            return jnp.logical_not(cry[2])

        def wbody(cry):
            g2, wb2, _ = cry
            need = (g2 * 16 - wb2) >= _W

            @pl.when(need)
            def _refill():
                _load_window(sp, sx, spbuf, sxbuf, wb2 + _W)

            wb3 = jnp.where(need, wb2 + _W, wb2)
            off = g2 * 16 - wb3
            a = spbuf[pl.ds(off, 16)]
            nxt = spbuf[pl.ds(off + 1, 16)]
            xv = sxbuf[pl.ds(off, 16)]
            keep = (a != nxt) & (a >= tile_base) & (a < tile_end)
            loc = jnp.clip(a - tile_base, 0, _T - 1)
            w_i = (loc * 43691) >> 22
            c_i = loc - w_i * 96
            plsc.store_scatter(tile, [w_i, c_i], xv, mask=keep)
            adv = (a[15] < tile_end) & (g2 < _NVREG - 1)
            g3 = jnp.where(adv, g2 + 1, g2)
            return (g3, wb3, jnp.logical_not(adv))

        g, wb, _ = lax.while_loop(wcond, wbody, (g, wb, jnp.bool_(False)))
        return g, wb

    g, wb = wb0 // 16, wb0
    pending = [None, None]
    for t in range(_NPLANES):
        p = t & 1
        if pending[p] is not None:
            pending[p].wait()
        tile_base = out_base + t * _T
        plane = wid * _NPLANES + t
        g, wb = merge_tile(tiles[p], tile_base, g, wb)
        pending[p] = pltpu.async_copy(
            tiles[p], out.at[plane // 224, plane % 224], sems[p]
        )
    for p in (0, 1):
        if pending[p] is not None:
            pending[p].wait()


_mesh = plsc.VectorSubcoreMesh(
    core_axis_name="c", subcore_axis_name="s", num_cores=_NC, num_subcores=_NS
)

_scatter_call = pl.kernel(
    _scatter_body,
    jax.ShapeDtypeStruct(_OUT_SHAPE, jnp.float32),
    mesh=_mesh,
    scratch_types=[
        pltpu.VMEM((16,), jnp.int32),
        pltpu.VMEM((_W + 16,), jnp.int32),
        pltpu.VMEM((_W,), jnp.float32),
        pltpu.VMEM((224, 96), jnp.float32),
        pltpu.VMEM((224, 96), jnp.float32),
        pltpu.SemaphoreType.DMA,
        pltpu.SemaphoreType.DMA,
    ],
    compiler_params=pltpu.CompilerParams(needs_layout_passes=False),
    name="unpool_scatter_sc",
)


def kernel(x, pos):
    xf = jnp.reshape(x, (-1,))
    posf = jnp.reshape(pos, (-1,)).astype(jnp.int32)
    sp, sx = lax.sort((posf, xf), dimension=0, is_stable=False, num_keys=1)
    return _scatter_call(sp, sx)
